# baseline (speedup 1.0000x reference)
import functools

import jax
import jax.numpy as jnp
from jax import lax
from jax.experimental import pallas as pl
from jax.experimental.pallas import tpu as pltpu
from jax.experimental.pallas import tpu_sc as plsc

_NC = 2
_NS = 16
_NW = _NC * _NS


def _make_sc_copy(n):
    per_w = n // _NW
    mesh = plsc.VectorSubcoreMesh(
        core_axis_name="c", subcore_axis_name="s",
        num_cores=_NC, num_subcores=_NS)

    @functools.partial(
        pl.kernel,
        out_type=[jax.ShapeDtypeStruct((n,), jnp.bfloat16)] * 2,
        mesh=mesh,
    )
    def sc_copy(k_hbm, v_hbm, ko_hbm, vo_hbm):
        wid = lax.axis_index("s") * _NC + lax.axis_index("c")
        base = wid * per_w
        sl = pl.ds(base, per_w)
        pltpu.sync_copy(k_hbm.at[sl], ko_hbm.at[sl])
        pltpu.sync_copy(v_hbm.at[sl], vo_hbm.at[sl])

    return sc_copy


def kernel(key_states, value_states, k_cache, v_cache, layer_idx):
    B, H, S, D = key_states.shape
    n = B * H * S * D
    k = lax.bitcast_convert_type(key_states, jnp.bfloat16).reshape(n)
    v = lax.bitcast_convert_type(value_states, jnp.bfloat16).reshape(n)
    ko, vo = _make_sc_copy(n)(k, v)
    ko = lax.bitcast_convert_type(ko.reshape(B, H, S, D), jnp.float16)
    vo = lax.bitcast_convert_type(vo.reshape(B, H, S, D), jnp.float16)
    return ko, vo


# speedup vs baseline: 12.0306x; 12.0306x over previous
import functools

import jax
import jax.numpy as jnp
from jax import lax
from jax.experimental import pallas as pl
from jax.experimental.pallas import tpu as pltpu
from jax.experimental.pallas import tpu_sc as plsc

_NC = 2
_NS = 16
_NW = _NC * _NS
_CH = 65536
_NB = 3


def _make_sc_copy(n):
    per_w = n // _NW
    chunks_per_tensor = per_w // _CH
    mesh = plsc.VectorSubcoreMesh(
        core_axis_name="c", subcore_axis_name="s",
        num_cores=_NC, num_subcores=_NS)

    @functools.partial(
        pl.kernel,
        out_type=[jax.ShapeDtypeStruct((n,), jnp.bfloat16)] * 2,
        mesh=mesh,
        scratch_types=(
            [pltpu.VMEM((_CH,), jnp.bfloat16)] * _NB
            + [pltpu.SemaphoreType.DMA] * (2 * _NB)
        ),
    )
    def sc_copy(k_hbm, v_hbm, ko_hbm, vo_hbm,
                b0, b1, b2, si0, si1, si2, so0, so1, so2):
        bufs = (b0, b1, b2)
        sin = (si0, si1, si2)
        sout = (so0, so1, so2)
        wid = lax.axis_index("s") * _NC + lax.axis_index("c")
        base = wid * per_w

        jobs = []
        for src, dst in ((k_hbm, ko_hbm), (v_hbm, vo_hbm)):
            for c in range(chunks_per_tensor):
                jobs.append((src, dst, c * _CH))
        ins, outs = [], []
        for j, (src, dst, off) in enumerate(jobs):
            b = j % _NB
            sl = pl.ds(base + off, _CH)
            ins.append(pltpu.make_async_copy(src.at[sl], bufs[b], sin[b]))
            outs.append(pltpu.make_async_copy(bufs[b], dst.at[sl], sout[b]))

        nj = len(jobs)
        for j in range(min(_NB, nj)):
            ins[j].start()
        for j in range(nj):
            ins[j].wait()
            outs[j].start()
            nxt = j + _NB
            if nxt < nj:
                outs[j].wait()
                ins[nxt].start()
        for j in range(max(0, nj - _NB), nj):
            outs[j].wait()

    return sc_copy


def kernel(key_states, value_states, k_cache, v_cache, layer_idx):
    B, H, S, D = key_states.shape
    n = B * H * S * D
    k = lax.bitcast_convert_type(key_states, jnp.bfloat16).reshape(n)
    v = lax.bitcast_convert_type(value_states, jnp.bfloat16).reshape(n)
    ko, vo = _make_sc_copy(n)(k, v)
    ko = lax.bitcast_convert_type(ko.reshape(B, H, S, D), jnp.float16)
    vo = lax.bitcast_convert_type(vo.reshape(B, H, S, D), jnp.float16)
    return ko, vo
